# baseline (speedup 1.0000x reference)
import functools

import jax
import jax.numpy as jnp
from jax import lax
from jax.experimental import pallas as pl
from jax.experimental.pallas import tpu as pltpu
from jax.experimental.pallas import tpu_sc as plsc

_N = 65536
_C_IN = 144 * 3
_C_OUT = 118 * 3
_LANES = 16
_NC, _NS = 2, 16
_NW = _NC * _NS
_ROWS_PER_W = _N // _NW
_BB = 128
_NBLK = _ROWS_PER_W // _BB
_GRP_OFF = tuple(min(g * _LANES, _C_OUT - _LANES)
                 for g in range((_C_OUT + _LANES - 1) // _LANES))
_NGRP = len(_GRP_OFF)

_mesh = plsc.VectorSubcoreMesh(core_axis_name="c", subcore_axis_name="s")


@functools.partial(
    pl.kernel,
    out_type=jax.ShapeDtypeStruct((_N, _C_OUT), jnp.float32),
    mesh=_mesh,
    scratch_types=[
        pltpu.VMEM((_NGRP * _LANES,), jnp.int32),
        pltpu.VMEM((_BB, _C_IN), jnp.float32),
        pltpu.VMEM((_BB, _C_OUT), jnp.float32),
    ],
    compiler_params=pltpu.CompilerParams(use_tc_tiling_on_sc=False,
                                         needs_layout_passes=False),
)
def _sc_gather(x_hbm, cols_hbm, out_hbm, cols_v, in_v, out_v):
    wid = lax.axis_index("s") * _NC + lax.axis_index("c")
    row0 = wid * _ROWS_PER_W
    pltpu.sync_copy(cols_hbm, cols_v)
    cvs = [cols_v[pl.ds(g * _LANES, _LANES)] for g in range(_NGRP)]

    def block_body(blk, carry):
        r0 = row0 + blk * _BB
        pltpu.sync_copy(x_hbm.at[pl.ds(r0, _BB), :], in_v)

        def row_body(b, c):
            bvec = lax.broadcast(b, (_LANES,))
            for og, cv in zip(_GRP_OFF, cvs):
                out_v[b, pl.ds(og, _LANES)] = plsc.load_gather(
                    in_v, [bvec, cv])
            return c

        lax.fori_loop(0, _BB, row_body, 0)
        pltpu.sync_copy(out_v, out_hbm.at[pl.ds(r0, _BB), :])
        return carry

    lax.fori_loop(0, _NBLK, block_body, 0)


@jax.jit
def kernel(joints, joint_maps):
    x = joints.reshape(_N, _C_IN)
    jm3 = joint_maps.astype(jnp.int32) * 3
    offs = jnp.arange(_NGRP * _LANES, dtype=jnp.int32)
    src = jnp.array([og + l for og in _GRP_OFF for l in range(_LANES)],
                    dtype=jnp.int32)
    cols = jm3[src // 3] + src % 3
    del offs
    out = _sc_gather(x, cols)
    return out.reshape(_N, _C_OUT // 3, 3)


# speedup vs baseline: 1.1405x; 1.1405x over previous
import functools

import jax
import jax.numpy as jnp
from jax import lax
from jax.experimental import pallas as pl
from jax.experimental.pallas import tpu as pltpu
from jax.experimental.pallas import tpu_sc as plsc

_N = 65536
_C_IN = 144 * 3
_C_OUT = 118 * 3
_LANES = 16
_NC, _NS = 2, 16
_NW = _NC * _NS
_ROWS_PER_W = _N // _NW
_BB = 64
_NBLK = _ROWS_PER_W // _BB
_GRP_OFF = tuple(min(g * _LANES, _C_OUT - _LANES)
                 for g in range((_C_OUT + _LANES - 1) // _LANES))
_NGRP = len(_GRP_OFF)

_mesh = plsc.VectorSubcoreMesh(core_axis_name="c", subcore_axis_name="s")


@functools.partial(
    pl.kernel,
    out_type=jax.ShapeDtypeStruct((_N, _C_OUT), jnp.float32),
    mesh=_mesh,
    scratch_types=[
        pltpu.VMEM((_NGRP * _LANES,), jnp.int32),
        pltpu.VMEM((_BB, _C_IN), jnp.float32),
        pltpu.VMEM((_BB, _C_IN), jnp.float32),
        pltpu.VMEM((_BB, _C_OUT), jnp.float32),
        pltpu.VMEM((_BB, _C_OUT), jnp.float32),
        pltpu.SemaphoreType.DMA,
        pltpu.SemaphoreType.DMA,
        pltpu.SemaphoreType.DMA,
        pltpu.SemaphoreType.DMA,
    ],
    compiler_params=pltpu.CompilerParams(use_tc_tiling_on_sc=False,
                                         needs_layout_passes=False),
)
def _sc_gather(x_hbm, cols_hbm, out_hbm, cols_v,
               in0, in1, ou0, ou1, is0, is1, os0, os1):
    wid = lax.axis_index("s") * _NC + lax.axis_index("c")
    row0 = wid * _ROWS_PER_W
    pltpu.sync_copy(cols_hbm, cols_v)
    cvs = [cols_v[pl.ds(g * _LANES, _LANES)] for g in range(_NGRP)]
    ins, ous, isems, osems = (in0, in1), (ou0, ou1), (is0, is1), (os0, os1)

    def in_copy(blk, s):
        return pltpu.make_async_copy(
            x_hbm.at[pl.ds(row0 + blk * _BB, _BB), :], ins[s], isems[s])

    def out_copy(blk, s):
        return pltpu.make_async_copy(
            ous[s], out_hbm.at[pl.ds(row0 + blk * _BB, _BB), :], osems[s])

    in_copy(0, 0).start()
    in_copy(1, 1).start()

    def pair_body(p, carry):
        for s in (0, 1):
            blk = p * 2 + s
            in_copy(blk, s).wait()

            @pl.when(p > 0)
            def _():
                out_copy(blk - 2, s).wait()

            in_v, out_v = ins[s], ous[s]

            @plsc.parallel_loop(0, _BB, unroll=4)
            def _(b):
                bvec = lax.broadcast(b, (_LANES,))
                for og, cv in zip(_GRP_OFF, cvs):
                    out_v[b, pl.ds(og, _LANES)] = plsc.load_gather(
                        in_v, [bvec, cv])

            out_copy(blk, s).start()

            @pl.when(blk + 2 < _NBLK)
            def _():
                in_copy(blk + 2, s).start()
        return carry

    lax.fori_loop(0, _NBLK // 2, pair_body, 0)
    out_copy(_NBLK - 2, 0).wait()
    out_copy(_NBLK - 1, 1).wait()


@jax.jit
def kernel(joints, joint_maps):
    x = joints.reshape(_N, _C_IN)
    jm3 = joint_maps.astype(jnp.int32) * 3
    src = jnp.array([og + l for og in _GRP_OFF for l in range(_LANES)],
                    dtype=jnp.int32)
    cols = jm3[src // 3] + src % 3
    out = _sc_gather(x, cols)
    return out.reshape(_N, _C_OUT // 3, 3)
